# baseline (speedup 1.0000x reference)
import functools

import jax
import jax.numpy as jnp
from jax import lax
from jax.experimental import pallas as pl
from jax.experimental.pallas import tpu as pltpu, tpu_sc as plsc

N_ROWS = 100000
D = 256
B = 16384
MOM = 0.2

NC = 2
NS = 16
NW = NC * NS
R = 3136
R_LAST = N_ROWS - (NW - 1) * R
R16 = R
C = 128
CAP = ((R + C - 1) // C) * C
NCH = CAP // C
DV = D // 16

_SENT = 0x7FFFFFFF


def _take(v, idx):
  return jnp.take_along_axis(v, idx, axis=0)


def _splat0(v16):
  return _take(v16, jnp.zeros((16,), jnp.int32))


def _sc_body(fout_hbm, plab_hbm, feat_hbm, out_hbm, labels_v, claim, srcs,
             dstl, dstl3d, fbuf, gbuf, sem_a, sem_b, sem_c):
  wid = lax.axis_index("s") * NC + lax.axis_index("c")
  lo = wid * R
  hi = lo + R
  iota = lax.iota(jnp.int32, 16)
  nxt_idx = (iota + 1) & 15

  is_last = wid == NW - 1

  @pl.when(jnp.logical_not(is_last))
  def _():
    cp = pltpu.make_async_copy(
        feat_hbm.at[pl.ds(lo, R)], out_hbm.at[pl.ds(lo, R)], sem_c)
    cp.start()

  @pl.when(is_last)
  def _():
    cp = pltpu.make_async_copy(
        feat_hbm.at[pl.ds(lo, R_LAST)], out_hbm.at[pl.ds(lo, R_LAST)], sem_c)
    cp.start()

  pltpu.sync_copy(plab_hbm, labels_v)

  minus1 = jnp.full((16,), -1, jnp.int32)

  @pl.loop(0, R16 // 16)
  def _(k):
    claim[pl.ds(k * 16, 16)] = minus1

  @pl.loop(0, B // 16)
  def _(s):
    l = labels_v[pl.ds(s * 16, 16)]
    i = s * 16 + iota
    inr = (l >= lo) & (l < hi)
    comp = jnp.where(inr, (l << 14) | i, _SENT)
    sk, _ = plsc.sort_key_val(comp, comp)
    slab = sk >> 14
    nlab = _take(slab, nxt_idx)
    win = ((slab != nlab) | (iota == 15)) & (sk != _SENT)
    idx = jnp.where(win, slab - lo, 0)
    plsc.store_scatter(claim, (idx,), sk & 0x3FFF, mask=win)

  @pl.loop(0, R16 // 16, init_carry=jnp.int32(0))
  def count(k, cnt):
    c = claim[pl.ds(k * 16, 16)]
    m = c >= 0
    mi = jnp.where(m, jnp.int32(1), jnp.int32(0))
    cum = plsc.cumsum(mi)
    pos = cnt + cum - 1
    posw = jnp.where(m, pos, 0)
    plsc.store_scatter(srcs, (posw,), c, mask=m)
    plsc.store_scatter(dstl, (posw,), lo + k * 16 + iota, mask=m)
    return cnt + jnp.sum(mi)

  k_cnt = count

  @pl.when(k_cnt > 0)
  def _():
    kpad = ((k_cnt + C - 1) // C) * C
    s0 = _splat0(srcs[pl.ds(0, 16)])
    d0 = _splat0(dstl[pl.ds(0, 16)])

    @pl.loop(0, C // 16)
    def _(j):
      offs = k_cnt + j * 16 + iota
      mk = offs < kpad
      offw = jnp.where(mk, offs, 0)
      plsc.store_scatter(srcs, (offw,), s0, mask=mk)
      plsc.store_scatter(dstl, (offw,), d0, mask=mk)

  @pl.loop(0, CAP // 16)
  def _(k):
    v = dstl[pl.ds(k * 16, 16)]
    ch = k // (C // 16)
    off = (k - ch * (C // 16)) * 16
    dstl3d[ch, 0, pl.ds(off, 16)] = v

  nchunks = (k_cnt + C - 1) // C

  @pl.when(jnp.logical_not(is_last))
  def _():
    pltpu.make_async_copy(
        feat_hbm.at[pl.ds(lo, R)], out_hbm.at[pl.ds(lo, R)], sem_c).wait()

  @pl.when(is_last)
  def _():
    pltpu.make_async_copy(
        feat_hbm.at[pl.ds(lo, R_LAST)], out_hbm.at[pl.ds(lo, R_LAST)],
        sem_c).wait()

  @pl.loop(0, nchunks)
  def _(t):
    cp_f = pltpu.make_async_copy(
        fout_hbm.at[srcs.at[pl.ds(t * C, C)]], fbuf, sem_a)
    cp_f.start()
    cp_g = pltpu.make_async_copy(
        feat_hbm.at[dstl3d.at[t, 0]], gbuf, sem_b)
    cp_g.start()
    cp_f.wait()
    cp_g.wait()

    @pl.loop(0, C)
    def _(r):
      acc = jnp.zeros((16,), jnp.float32)
      for j in range(DV):
        g = gbuf[r, pl.ds(j * 16, 16)]
        f = fbuf[r, pl.ds(j * 16, 16)]
        m = MOM * g + (1.0 - MOM) * f
        fbuf[r, pl.ds(j * 16, 16)] = m
        acc = acc + m * m
      tot = _take(plsc.cumsum(acc), jnp.full((16,), 15, jnp.int32))
      bits = plsc.bitcast(tot, jnp.int32)
      y = plsc.bitcast(jnp.int32(0x5F3759DF) - (bits >> 1), jnp.float32)
      for _ in range(3):
        y = y * (1.5 - 0.5 * tot * y * y)
      for j in range(DV):
        fbuf[r, pl.ds(j * 16, 16)] = fbuf[r, pl.ds(j * 16, 16)] * y

    cp_o = pltpu.make_async_copy(fbuf, out_hbm.at[dstl3d.at[t, 0]], sem_a)
    cp_o.start()
    cp_o.wait()


def kernel(f_out, p_labels, features):
  mesh = plsc.VectorSubcoreMesh(
      core_axis_name="c", subcore_axis_name="s", num_cores=NC)
  run = pl.kernel(
      _sc_body,
      out_type=jax.ShapeDtypeStruct((N_ROWS, D), jnp.float32),
      mesh=mesh,
      compiler_params=pltpu.CompilerParams(needs_layout_passes=False),
      scratch_types=[
          pltpu.VMEM((B,), jnp.int32),
          pltpu.VMEM((R16,), jnp.int32),
          pltpu.VMEM((CAP + 16,), jnp.int32),
          pltpu.VMEM((CAP + 16,), jnp.int32),
          pltpu.VMEM((NCH, 1, C), jnp.int32),
          pltpu.VMEM((C, D), jnp.float32),
          pltpu.VMEM((C, D), jnp.float32),
          pltpu.SemaphoreType.DMA,
          pltpu.SemaphoreType.DMA,
          pltpu.SemaphoreType.DMA,
      ],
  )
  return run(f_out, p_labels, features)


# speedup vs baseline: 22.8246x; 22.8246x over previous
import jax
import jax.numpy as jnp
from jax import lax
from jax.experimental import pallas as pl
from jax.experimental.pallas import tpu as pltpu, tpu_sc as plsc

N_ROWS = 100000
D = 256
B = 16384
MOM = 0.2

NC = 2
NS = 16
NW = NC * NS
R = 3136
R16 = R
C = 96
CAP = ((R + C - 1) // C) * C
NCH = CAP // C
DV = D // 16

_SENT = 0x7FFFFFFF


def _take(v, idx):
  return jnp.take_along_axis(v, idx, axis=0)


def _splat0(v16):
  return _take(v16, jnp.zeros((16,), jnp.int32))


def _body_a(plab_hbm, srcs_hbm, dstl_hbm, nch_hbm, labels_v, claim, srcs,
            dstl, nch_v, sem):
  wid = lax.axis_index("s") * NC + lax.axis_index("c")
  lo = wid * R
  hi = lo + R
  iota = lax.iota(jnp.int32, 16)
  nxt_idx = (iota + 1) & 15

  pltpu.sync_copy(plab_hbm, labels_v)

  minus1 = jnp.full((16,), -1, jnp.int32)

  @pl.loop(0, R16 // 16)
  def _(k):
    claim[pl.ds(k * 16, 16)] = minus1

  @pl.loop(0, B // 16)
  def _(s):
    l = labels_v[pl.ds(s * 16, 16)]
    i = s * 16 + iota
    inr = (l >= lo) & (l < hi)
    comp = jnp.where(inr, (l << 14) | i, _SENT)
    sk, _ = plsc.sort_key_val(comp, comp)
    slab = sk >> 14
    nlab = _take(slab, nxt_idx)
    win = ((slab != nlab) | (iota == 15)) & (sk != _SENT)
    idx = jnp.where(win, slab - lo, 0)
    plsc.store_scatter(claim, (idx,), sk & 0x3FFF, mask=win)

  @pl.loop(0, R16 // 16, init_carry=jnp.int32(0))
  def count(k, cnt):
    c = claim[pl.ds(k * 16, 16)]
    m = c >= 0
    mi = jnp.where(m, jnp.int32(1), jnp.int32(0))
    cum = plsc.cumsum(mi)
    posw = jnp.where(m, cnt + cum - 1, 0)
    plsc.store_scatter(srcs, (posw,), c, mask=m)
    plsc.store_scatter(dstl, (posw,), lo + k * 16 + iota, mask=m)
    return cnt + jnp.sum(mi)

  k_cnt = count

  @pl.when(k_cnt > 0)
  def _():
    kpad = ((k_cnt + C - 1) // C) * C
    s0 = _splat0(srcs[pl.ds(0, 16)])
    d0 = _splat0(dstl[pl.ds(0, 16)])

    @pl.loop(0, C // 16)
    def _(j):
      offs = k_cnt + j * 16 + iota
      mk = offs < kpad
      offw = jnp.where(mk, offs, 0)
      plsc.store_scatter(srcs, (offw,), s0, mask=mk)
      plsc.store_scatter(dstl, (offw,), d0, mask=mk)

  nchunks = (k_cnt + C - 1) // C
  nch_v[pl.ds(0, 16)] = jnp.full((16,), 0, jnp.int32) + nchunks

  pltpu.sync_copy(srcs.at[pl.ds(0, CAP)], srcs_hbm.at[pl.ds(wid * CAP, CAP)])
  pltpu.sync_copy(dstl.at[pl.ds(0, CAP)], dstl_hbm.at[pl.ds(wid * CAP, CAP)])
  pltpu.sync_copy(nch_v, nch_hbm.at[pl.ds(wid * 16, 16)])


def _body_b(srcs_hbm, dstl_hbm, nch_hbm, fout_hbm, feat_hbm, out_ref,
            srcs, dstl, dstl3d, nch_v, fbuf0, gbuf0, fbuf1, gbuf1,
            sem_g0, sem_g1, sem_s0, sem_s1):
  wid = lax.axis_index("s") * NC + lax.axis_index("c")

  pltpu.sync_copy(srcs_hbm.at[pl.ds(wid * CAP, CAP)], srcs.at[pl.ds(0, CAP)])
  pltpu.sync_copy(dstl_hbm.at[pl.ds(wid * CAP, CAP)], dstl.at[pl.ds(0, CAP)])
  pltpu.sync_copy(nch_hbm.at[pl.ds(wid * 16, 16)], nch_v)
  nchunks = nch_v[pl.ds(0, 16)][0]

  @pl.loop(0, CAP // 16)
  def _(k):
    v = dstl[pl.ds(k * 16, 16)]
    ch = k // (C // 16)
    off = (k - ch * (C // 16)) * 16
    dstl3d[ch, 0, pl.ds(off, 16)] = v

  bufs = ((fbuf0, gbuf0, sem_g0, sem_s0), (fbuf1, gbuf1, sem_g1, sem_s1))

  def start_gather(t, fb, gb, sg):
    pltpu.make_async_copy(
        fout_hbm.at[srcs.at[pl.ds(t * C, C)]], fb, sg).start()
    pltpu.make_async_copy(feat_hbm.at[dstl3d.at[t, 0]], gb, sg).start()

  def wait_gather(t, fb, gb, sg):
    pltpu.make_async_copy(
        fout_hbm.at[srcs.at[pl.ds(t * C, C)]], fb, sg).wait()
    pltpu.make_async_copy(feat_hbm.at[dstl3d.at[t, 0]], gb, sg).wait()

  @pl.when(nchunks > 0)
  def _():
    start_gather(0, fbuf0, gbuf0, sem_g0)

  @pl.loop(0, nchunks)
  def _(t):
    par = t & 1

    for p in range(2):
      fb, gb, sg, ss = bufs[p]

      @pl.when(par == p)
      def _():
        @pl.when(t >= 1)
        def _():
          ofb, _, _, oss = bufs[1 - p]
          pltpu.make_async_copy(
              ofb, out_ref.at[dstl3d.at[t - 1, 0]], oss).wait()

        @pl.when(t + 1 < nchunks)
        def _():
          nfb, ngb, nsg, _ = bufs[1 - p]
          start_gather(t + 1, nfb, ngb, nsg)

        wait_gather(t, fb, gb, sg)

        @pl.loop(0, C)
        def _(r):
          acc = jnp.zeros((16,), jnp.float32)
          m = []
          for j in range(DV):
            g = gb[r, pl.ds(j * 16, 16)]
            f = fb[r, pl.ds(j * 16, 16)]
            mj = MOM * g + (1.0 - MOM) * f
            m.append(mj)
            acc = acc + mj * mj
          tot = _take(plsc.cumsum(acc), jnp.full((16,), 15, jnp.int32))
          bits = plsc.bitcast(tot, jnp.int32)
          y = plsc.bitcast(jnp.int32(0x5F3759DF) - (bits >> 1), jnp.float32)
          for _ in range(3):
            y = y * (1.5 - 0.5 * tot * y * y)
          for j in range(DV):
            fb[r, pl.ds(j * 16, 16)] = m[j] * y

        pltpu.make_async_copy(fb, out_ref.at[dstl3d.at[t, 0]], ss).start()

  @pl.when(nchunks > 0)
  def _():
    for p in range(2):
      fb, _, _, ss = bufs[p]

      @pl.when((nchunks - 1) & 1 == p)
      def _():
        pltpu.make_async_copy(
            fb, out_ref.at[dstl3d.at[nchunks - 1, 0]], ss).wait()


def kernel(f_out, p_labels, features):
  mesh = plsc.VectorSubcoreMesh(
      core_axis_name="c", subcore_axis_name="s", num_cores=NC)
  cp = pltpu.CompilerParams(needs_layout_passes=False)

  run_a = pl.kernel(
      _body_a,
      out_type=(
          jax.ShapeDtypeStruct((NW * CAP,), jnp.int32),
          jax.ShapeDtypeStruct((NW * CAP,), jnp.int32),
          jax.ShapeDtypeStruct((NW * 16,), jnp.int32),
      ),
      mesh=mesh,
      compiler_params=cp,
      scratch_types=[
          pltpu.VMEM((B,), jnp.int32),
          pltpu.VMEM((R16,), jnp.int32),
          pltpu.VMEM((CAP + 16,), jnp.int32),
          pltpu.VMEM((CAP + 16,), jnp.int32),
          pltpu.VMEM((16,), jnp.int32),
          pltpu.SemaphoreType.DMA,
      ],
  )
  srcs_hbm, dstl_hbm, nch_hbm = run_a(p_labels)

  out_ref = jax.new_ref(features)
  run_b = pl.kernel(
      _body_b,
      out_type=(),
      mesh=mesh,
      compiler_params=cp,
      scratch_types=[
          pltpu.VMEM((CAP + 16,), jnp.int32),
          pltpu.VMEM((CAP + 16,), jnp.int32),
          pltpu.VMEM((NCH, 1, C), jnp.int32),
          pltpu.VMEM((16,), jnp.int32),
          pltpu.VMEM((C, D), jnp.float32),
          pltpu.VMEM((C, D), jnp.float32),
          pltpu.VMEM((C, D), jnp.float32),
          pltpu.VMEM((C, D), jnp.float32),
          pltpu.SemaphoreType.DMA,
          pltpu.SemaphoreType.DMA,
          pltpu.SemaphoreType.DMA,
          pltpu.SemaphoreType.DMA,
      ],
  )
  run_b(srcs_hbm, dstl_hbm, nch_hbm, f_out, features, out_ref)
  return out_ref[...]


# speedup vs baseline: 26.6917x; 1.1694x over previous
import jax
import jax.numpy as jnp
from jax import lax
from jax.experimental import pallas as pl
from jax.experimental.pallas import tpu as pltpu, tpu_sc as plsc

N_ROWS = 100000
D = 256
B = 16384
MOM = 0.2

NC = 2
NS = 16
NW = NC * NS
R = 3136
R16 = R
C = 96
CAP = ((R + C - 1) // C) * C
NCH = CAP // C
DV = D // 16

_SENT = 0x7FFFFFFF


def _take(v, idx):
  return jnp.take_along_axis(v, idx, axis=0)


def _splat0(v16):
  return _take(v16, jnp.zeros((16,), jnp.int32))


def _body_a(plab_hbm, srcs_hbm, dstl_hbm, nch_hbm, labels_v, claim, srcs,
            dstl, nch_v, sem):
  wid = lax.axis_index("s") * NC + lax.axis_index("c")
  lo = wid * R
  hi = lo + R
  iota = lax.iota(jnp.int32, 16)
  nxt_idx = (iota + 1) & 15

  pltpu.sync_copy(plab_hbm, labels_v)

  minus1 = jnp.full((16,), -1, jnp.int32)

  @pl.loop(0, R16 // 16)
  def _(k):
    claim[pl.ds(k * 16, 16)] = minus1

  @pl.loop(0, B // 16)
  def _(s):
    l = labels_v[pl.ds(s * 16, 16)]
    i = s * 16 + iota
    inr = (l >= lo) & (l < hi)
    comp = jnp.where(inr, (l << 14) | i, _SENT)
    sk, _ = plsc.sort_key_val(comp, comp)
    slab = sk >> 14
    nlab = _take(slab, nxt_idx)
    win = ((slab != nlab) | (iota == 15)) & (sk != _SENT)
    idx = jnp.where(win, slab - lo, 0)
    plsc.store_scatter(claim, (idx,), sk & 0x3FFF, mask=win)

  @pl.loop(0, R16 // 16, init_carry=jnp.int32(0))
  def count(k, cnt):
    c = claim[pl.ds(k * 16, 16)]
    m = c >= 0
    mi = jnp.where(m, jnp.int32(1), jnp.int32(0))
    cum = plsc.cumsum(mi)
    posw = jnp.where(m, cnt + cum - 1, 0)
    plsc.store_scatter(srcs, (posw,), c, mask=m)
    plsc.store_scatter(dstl, (posw,), lo + k * 16 + iota, mask=m)
    return cnt + jnp.sum(mi)

  k_cnt = count

  @pl.when(k_cnt > 0)
  def _():
    kpad = ((k_cnt + C - 1) // C) * C

    @pl.loop(0, C // 16)
    def _(j):
      offs = k_cnt + j * 16 + iota
      mk = offs < kpad
      offw = jnp.where(mk, offs, 0)
      srcoff = jnp.maximum(offw - C, 0)
      plsc.store_scatter(srcs, (offw,), plsc.load_gather(srcs, (srcoff,)),
                         mask=mk)
      plsc.store_scatter(dstl, (offw,), plsc.load_gather(dstl, (srcoff,)),
                         mask=mk)

  nchunks = (k_cnt + C - 1) // C
  nch_v[pl.ds(0, 16)] = jnp.full((16,), 0, jnp.int32) + nchunks

  pltpu.sync_copy(srcs.at[pl.ds(0, CAP)], srcs_hbm.at[pl.ds(wid * CAP, CAP)])
  pltpu.sync_copy(dstl.at[pl.ds(0, CAP)], dstl_hbm.at[pl.ds(wid * CAP, CAP)])
  pltpu.sync_copy(nch_v, nch_hbm.at[pl.ds(wid * 16, 16)])


def _body_b(srcs_hbm, dstl_hbm, nch_hbm, fout_hbm, feat_hbm, out_ref,
            srcs, dstl, dstl3d, nch_v, fbuf0, gbuf0, fbuf1, gbuf1,
            sem_g0, sem_g1, sem_s0, sem_s1):
  wid = lax.axis_index("s") * NC + lax.axis_index("c")

  pltpu.sync_copy(srcs_hbm.at[pl.ds(wid * CAP, CAP)], srcs.at[pl.ds(0, CAP)])
  pltpu.sync_copy(dstl_hbm.at[pl.ds(wid * CAP, CAP)], dstl.at[pl.ds(0, CAP)])
  pltpu.sync_copy(nch_hbm.at[pl.ds(wid * 16, 16)], nch_v)
  nchunks = nch_v[pl.ds(0, 16)][0]

  @pl.loop(0, CAP // 16)
  def _(k):
    v = dstl[pl.ds(k * 16, 16)]
    ch = k // (C // 16)
    off = (k - ch * (C // 16)) * 16
    dstl3d[ch, 0, pl.ds(off, 16)] = v

  bufs = ((fbuf0, gbuf0, sem_g0, sem_s0), (fbuf1, gbuf1, sem_g1, sem_s1))

  def start_gather(t, fb, gb, sg):
    pltpu.make_async_copy(
        fout_hbm.at[srcs.at[pl.ds(t * C, C)]], fb, sg).start()
    pltpu.make_async_copy(feat_hbm.at[dstl3d.at[t, 0]], gb, sg).start()

  def wait_gather(t, fb, gb, sg):
    pltpu.make_async_copy(
        fout_hbm.at[srcs.at[pl.ds(t * C, C)]], fb, sg).wait()
    pltpu.make_async_copy(feat_hbm.at[dstl3d.at[t, 0]], gb, sg).wait()

  @pl.when(nchunks > 0)
  def _():
    start_gather(0, fbuf0, gbuf0, sem_g0)

  @pl.loop(0, nchunks)
  def _(t):
    par = t & 1

    for p in range(2):
      fb, gb, sg, ss = bufs[p]

      @pl.when(par == p)
      def _():
        @pl.when(t >= 1)
        def _():
          ofb, _, _, oss = bufs[1 - p]
          pltpu.make_async_copy(
              ofb, out_ref.at[dstl3d.at[t - 1, 0]], oss).wait()

        @pl.when(t + 1 < nchunks)
        def _():
          nfb, ngb, nsg, _ = bufs[1 - p]
          start_gather(t + 1, nfb, ngb, nsg)

        wait_gather(t, fb, gb, sg)

        @pl.loop(0, C, unroll=2)
        def _(r):
          acc = jnp.zeros((16,), jnp.float32)
          m = []
          for j in range(DV):
            g = gb[r, pl.ds(j * 16, 16)]
            f = fb[r, pl.ds(j * 16, 16)]
            mj = MOM * g + (1.0 - MOM) * f
            m.append(mj)
            acc = acc + mj * mj
          tot = _take(plsc.cumsum(acc), jnp.full((16,), 15, jnp.int32))
          bits = plsc.bitcast(tot, jnp.int32)
          y = plsc.bitcast(jnp.int32(0x5F3759DF) - (bits >> 1), jnp.float32)
          for _ in range(2):
            y = y * (1.5 - 0.5 * tot * y * y)
          for j in range(DV):
            fb[r, pl.ds(j * 16, 16)] = m[j] * y

        pltpu.make_async_copy(fb, out_ref.at[dstl3d.at[t, 0]], ss).start()

  @pl.when(nchunks > 0)
  def _():
    for p in range(2):
      fb, _, _, ss = bufs[p]

      @pl.when((nchunks - 1) & 1 == p)
      def _():
        pltpu.make_async_copy(
            fb, out_ref.at[dstl3d.at[nchunks - 1, 0]], ss).wait()


def kernel(f_out, p_labels, features):
  mesh = plsc.VectorSubcoreMesh(
      core_axis_name="c", subcore_axis_name="s", num_cores=NC)
  cp = pltpu.CompilerParams(needs_layout_passes=False)

  run_a = pl.kernel(
      _body_a,
      out_type=(
          jax.ShapeDtypeStruct((NW * CAP,), jnp.int32),
          jax.ShapeDtypeStruct((NW * CAP,), jnp.int32),
          jax.ShapeDtypeStruct((NW * 16,), jnp.int32),
      ),
      mesh=mesh,
      compiler_params=cp,
      scratch_types=[
          pltpu.VMEM((B,), jnp.int32),
          pltpu.VMEM((R16,), jnp.int32),
          pltpu.VMEM((CAP + 16,), jnp.int32),
          pltpu.VMEM((CAP + 16,), jnp.int32),
          pltpu.VMEM((16,), jnp.int32),
          pltpu.SemaphoreType.DMA,
      ],
  )
  srcs_hbm, dstl_hbm, nch_hbm = run_a(p_labels)

  out_ref = jax.new_ref(features)
  run_b = pl.kernel(
      _body_b,
      out_type=(),
      mesh=mesh,
      compiler_params=cp,
      scratch_types=[
          pltpu.VMEM((CAP + 16,), jnp.int32),
          pltpu.VMEM((CAP + 16,), jnp.int32),
          pltpu.VMEM((NCH, 1, C), jnp.int32),
          pltpu.VMEM((16,), jnp.int32),
          pltpu.VMEM((C, D), jnp.float32),
          pltpu.VMEM((C, D), jnp.float32),
          pltpu.VMEM((C, D), jnp.float32),
          pltpu.VMEM((C, D), jnp.float32),
          pltpu.SemaphoreType.DMA,
          pltpu.SemaphoreType.DMA,
          pltpu.SemaphoreType.DMA,
          pltpu.SemaphoreType.DMA,
      ],
  )
  run_b(srcs_hbm, dstl_hbm, nch_hbm, f_out, features, out_ref)
  return out_ref[...]
